# Initial kernel scaffold; baseline (speedup 1.0000x reference)
import jax
import jax.numpy as jnp
from jax.experimental import pallas as pl


def kernel(sequence, ngrams, word_table, ngram_table, W1, b1, W2, b2):
    raise NotImplementedError("write your pallas kernel here")


# speedup vs baseline: 5.0150x; 5.0150x over previous
import functools

import jax
import jax.numpy as jnp
from jax import lax
from jax.experimental import pallas as pl
from jax.experimental.pallas import tpu as pltpu
from jax.experimental.pallas import tpu_sc as plsc

BATCH = 4096
SEQ = 200
DIM = 32
HALF = SEQ // 2

NC = 2
NS = 16
NW = NC * NS
ROWS_PER_W = BATCH // NW

_mesh = plsc.VectorSubcoreMesh(core_axis_name="c", subcore_axis_name="s")


@functools.partial(
    pl.kernel,
    out_type=[
        jax.ShapeDtypeStruct((BATCH, DIM), jnp.float32),
        jax.ShapeDtypeStruct((BATCH, DIM), jnp.float32),
    ],
    mesh=_mesh,
    scratch_types=[
        pltpu.VMEM((2, HALF), jnp.int32),
        pltpu.VMEM((2, HALF), jnp.int32),
        pltpu.VMEM((SEQ, DIM), jnp.float32),
        pltpu.VMEM((SEQ, DIM), jnp.float32),
        pltpu.VMEM((ROWS_PER_W, DIM), jnp.float32),
        pltpu.VMEM((ROWS_PER_W, DIM), jnp.float32),
        pltpu.SemaphoreType.DMA,
    ],
    compiler_params=pltpu.CompilerParams(use_tc_tiling_on_sc=False),
)
def _pool(seq_hbm, ng_hbm, wtab_hbm, ntab_hbm, wout_hbm, nout_hbm,
          widx, nidx, wrows, nrows, wacc, nacc, sem):
    wid = lax.axis_index("s") * NC + lax.axis_index("c")
    base = wid * ROWS_PER_W
    inv = 1.0 / SEQ

    def _reduce_mean(rows, acc_ref, r):
        def blk(jj, carry):
            a0, a1, b0, b1 = carry
            j = jj * 8
            for u in range(0, 8, 2):
                a0 = a0 + rows[j + u, pl.ds(0, 16)]
                a1 = a1 + rows[j + u, pl.ds(16, 16)]
                b0 = b0 + rows[j + u + 1, pl.ds(0, 16)]
                b1 = b1 + rows[j + u + 1, pl.ds(16, 16)]
            return a0, a1, b0, b1

        z = jnp.zeros((16,), jnp.float32)
        a0, a1, b0, b1 = lax.fori_loop(0, SEQ // 8, blk, (z, z, z, z))
        acc_ref[r, pl.ds(0, 16)] = (a0 + b0) * inv
        acc_ref[r, pl.ds(16, 16)] = (a1 + b1) * inv

    def body(r, carry):
        row = base + r
        pltpu.sync_copy(seq_hbm.at[row], widx)
        pltpu.sync_copy(ng_hbm.at[row], nidx)
        cw0 = pltpu.async_copy(wtab_hbm.at[widx.at[0]], wrows.at[pl.ds(0, HALF)], sem)
        cw1 = pltpu.async_copy(wtab_hbm.at[widx.at[1]], wrows.at[pl.ds(HALF, HALF)], sem)
        cn0 = pltpu.async_copy(ntab_hbm.at[nidx.at[0]], nrows.at[pl.ds(0, HALF)], sem)
        cn1 = pltpu.async_copy(ntab_hbm.at[nidx.at[1]], nrows.at[pl.ds(HALF, HALF)], sem)
        cw0.wait()
        cw1.wait()
        _reduce_mean(wrows, wacc, r)
        cn0.wait()
        cn1.wait()
        _reduce_mean(nrows, nacc, r)
        return carry

    lax.fori_loop(0, ROWS_PER_W, body, 0)
    pltpu.sync_copy(wacc, wout_hbm.at[pl.ds(base, ROWS_PER_W)])
    pltpu.sync_copy(nacc, nout_hbm.at[pl.ds(base, ROWS_PER_W)])


def _mlp_body(x1_ref, x2_ref, w1a_ref, w1b_ref, b1_ref, w2_ref, b2_ref, o_ref):
    h = (
        jnp.dot(x1_ref[...], w1a_ref[...], preferred_element_type=jnp.float32)
        + jnp.dot(x2_ref[...], w1b_ref[...], preferred_element_type=jnp.float32)
        + b1_ref[...]
    )
    h = jnp.maximum(h, 0.0)
    o_ref[...] = (
        jnp.dot(h, w2_ref[...], preferred_element_type=jnp.float32) + b2_ref[...]
    )


_mlp = pl.pallas_call(
    _mlp_body,
    out_shape=jax.ShapeDtypeStruct((BATCH, 10), jnp.float32),
)


def kernel(sequence, ngrams, word_table, ngram_table, W1, b1, W2, b2):
    seq3 = sequence.astype(jnp.int32).reshape(BATCH, 2, HALF)
    ng3 = ngrams.astype(jnp.int32).reshape(BATCH, 2, HALF)
    embs, ngram_embs = _pool(seq3, ng3, word_table, ngram_table)
    w1a = W1[:, :DIM].T
    w1b = W1[:, DIM:].T
    out = _mlp(embs, ngram_embs, w1a, w1b,
               b1.reshape(1, -1), W2.T, b2.reshape(1, -1))
    return out


# speedup vs baseline: 6.6814x; 1.3323x over previous
import functools

import jax
import jax.numpy as jnp
from jax import lax
from jax.experimental import pallas as pl
from jax.experimental.pallas import tpu as pltpu
from jax.experimental.pallas import tpu_sc as plsc

BATCH = 4096
SEQ = 200
DIM = 32
CH0 = 104
CH1 = SEQ - CH0

NC = 2
NS = 16
NW = NC * NS
ROWS_PER_W = BATCH // NW

_mesh = plsc.VectorSubcoreMesh(core_axis_name="c", subcore_axis_name="s")

TBLK = 4096


def _xpose_body(x_ref, o_ref):
    o_ref[...] = x_ref[...].T.reshape(TBLK // 4, 128)


def _linearize(table_t, v):
    grid = -(-v // TBLK)
    wide = pl.pallas_call(
        _xpose_body,
        grid=(grid,),
        in_specs=[pl.BlockSpec((DIM, TBLK), lambda g: (0, g))],
        out_specs=pl.BlockSpec((TBLK // 4, 128), lambda g: (g, 0)),
        out_shape=jax.ShapeDtypeStruct((v * DIM // 128, 128), jnp.float32),
    )(table_t)
    return wide.reshape(v, DIM)


def _pool_body(idx_hbm, tab_hbm, out_hbm, idx_a, idx_b, rows_a, rows_b,
               acc, si_a, si_b, sg_a, sg_b):
    wid = lax.axis_index("s") * NC + lax.axis_index("c")
    base = wid * ROWS_PER_W
    inv = 1.0 / SEQ

    def issue_idx(row, idx_ref, sem):
        return pltpu.async_copy(idx_hbm.at[row], idx_ref, sem)

    def wait_idx(row, idx_ref, sem):
        pltpu.make_async_copy(idx_hbm.at[row], idx_ref, sem).wait()

    def issue_gathers(idx_ref, rows_ref, sem):
        pltpu.async_copy(tab_hbm.at[idx_ref.at[pl.ds(0, CH0)]],
                         rows_ref.at[pl.ds(0, CH0)], sem)
        pltpu.async_copy(tab_hbm.at[idx_ref.at[pl.ds(CH0, CH1)]],
                         rows_ref.at[pl.ds(CH0, CH1)], sem)

    def wait_gathers(idx_ref, rows_ref, sem):
        pltpu.make_async_copy(tab_hbm.at[idx_ref.at[pl.ds(0, CH0)]],
                              rows_ref.at[pl.ds(0, CH0)], sem).wait()
        pltpu.make_async_copy(tab_hbm.at[idx_ref.at[pl.ds(CH0, CH1)]],
                              rows_ref.at[pl.ds(CH0, CH1)], sem).wait()

    def reduce_mean(rows_ref, r):
        def blk(jj, carry):
            a0, a1, b0, b1 = carry
            j = jj * 8
            for u in range(0, 8, 2):
                a0 = a0 + rows_ref[j + u, pl.ds(0, 16)]
                a1 = a1 + rows_ref[j + u, pl.ds(16, 16)]
                b0 = b0 + rows_ref[j + u + 1, pl.ds(0, 16)]
                b1 = b1 + rows_ref[j + u + 1, pl.ds(16, 16)]
            return a0, a1, b0, b1

        z = jnp.zeros((16,), jnp.float32)
        a0, a1, b0, b1 = lax.fori_loop(0, SEQ // 8, blk, (z, z, z, z))
        acc[r, pl.ds(0, 16)] = (a0 + b0) * inv
        acc[r, pl.ds(16, 16)] = (a1 + b1) * inv

    c0 = issue_idx(base, idx_a, si_a)
    c0.wait()
    issue_gathers(idx_a, rows_a, sg_a)
    issue_idx(base + 1, idx_b, si_b)

    def body(rr, carry):
        r0 = base + 2 * rr
        wait_idx(r0 + 1, idx_b, si_b)
        issue_gathers(idx_b, rows_b, sg_b)
        wait_gathers(idx_a, rows_a, sg_a)
        issue_idx(r0 + 2, idx_a, si_a)
        reduce_mean(rows_a, 2 * rr)
        wait_idx(r0 + 2, idx_a, si_a)
        issue_gathers(idx_a, rows_a, sg_a)
        wait_gathers(idx_b, rows_b, sg_b)
        issue_idx(r0 + 3, idx_b, si_b)
        reduce_mean(rows_b, 2 * rr + 1)
        return carry

    lax.fori_loop(0, ROWS_PER_W // 2 - 1, body, 0)
    r0 = base + ROWS_PER_W - 2
    wait_idx(r0 + 1, idx_b, si_b)
    issue_gathers(idx_b, rows_b, sg_b)
    wait_gathers(idx_a, rows_a, sg_a)
    reduce_mean(rows_a, ROWS_PER_W - 2)
    wait_gathers(idx_b, rows_b, sg_b)
    reduce_mean(rows_b, ROWS_PER_W - 1)

    pltpu.sync_copy(acc, out_hbm.at[pl.ds(base, ROWS_PER_W)])


_pool = pl.kernel(
    _pool_body,
    out_type=jax.ShapeDtypeStruct((BATCH, DIM), jnp.float32),
    mesh=_mesh,
    scratch_types=[
        pltpu.VMEM((SEQ,), jnp.int32),
        pltpu.VMEM((SEQ,), jnp.int32),
        pltpu.VMEM((SEQ, DIM), jnp.float32),
        pltpu.VMEM((SEQ, DIM), jnp.float32),
        pltpu.VMEM((ROWS_PER_W, DIM), jnp.float32),
        pltpu.SemaphoreType.DMA,
        pltpu.SemaphoreType.DMA,
        pltpu.SemaphoreType.DMA,
        pltpu.SemaphoreType.DMA,
    ],
    compiler_params=pltpu.CompilerParams(use_tc_tiling_on_sc=False),
)


def _mlp_body(x1_ref, x2_ref, w1a_ref, w1b_ref, b1_ref, w2_ref, b2_ref, o_ref):
    h = (
        jnp.dot(x1_ref[...], w1a_ref[...], preferred_element_type=jnp.float32)
        + jnp.dot(x2_ref[...], w1b_ref[...], preferred_element_type=jnp.float32)
        + b1_ref[...]
    )
    h = jnp.maximum(h, 0.0)
    o_ref[...] = (
        jnp.dot(h, w2_ref[...], preferred_element_type=jnp.float32) + b2_ref[...]
    )


_mlp = pl.pallas_call(
    _mlp_body,
    out_shape=jax.ShapeDtypeStruct((BATCH, 10), jnp.float32),
)


def kernel(sequence, ngrams, word_table, ngram_table, W1, b1, W2, b2):
    seq = sequence.astype(jnp.int32)
    ng = ngrams.astype(jnp.int32)
    wlin = word_table
    nlin = ngram_table
    embs = _pool(seq, wlin)
    ngram_embs = _pool(ng, nlin)
    w1a = W1[:, :DIM].T
    w1b = W1[:, DIM:].T
    out = _mlp(embs, ngram_embs, w1a, w1b,
               b1.reshape(1, -1), W2.T, b2.reshape(1, -1))
    return out
